# Initial kernel scaffold; baseline (speedup 1.0000x reference)
import jax
import jax.numpy as jnp
from jax.experimental import pallas as pl


def kernel(h, edge_index, W1, b1, W2, b2):
    raise NotImplementedError("write your pallas kernel here")


# speedup vs baseline: 13.6297x; 13.6297x over previous
import jax
import jax.numpy as jnp
from jax import lax
from jax.experimental import pallas as pl
from jax.experimental.pallas import tpu as pltpu
from jax.experimental.pallas import tpu_sc as plsc

_N = 10000
_E = 320000
_D = 128
_L = 16
_NC = 2
_NS = 16
_NW = _NC * _NS
_C = 128
_NCHUNK = _E // _C
_CHUNK_PER_W = _NCHUNK // _NW
_CHUNK_REM = _NCHUNK - _CHUNK_PER_W * _NW
_RPT_STRIDE = 624
_RPT_SPAN = 640
_ZCH = 128
_NIDX = 4
_DROWS = 80
_DCOL = 125

_mesh = plsc.VectorSubcoreMesh(core_axis_name="c", subcore_axis_name="s")


def _worker_ranges():
    c = lax.axis_index("c")
    s = lax.axis_index("s")
    wid = c * _NS + s
    start = wid * _CHUNK_PER_W + jnp.minimum(wid, _CHUNK_REM)
    nch = jnp.where(wid < _CHUNK_REM, _CHUNK_PER_W + 1, _CHUNK_PER_W)
    return c, s, start, nch


def _make_agg_body(with_deg, nbuf):
    ngrp = 4 if nbuf == 2 else 12

    def _body(*refs):
        it = iter(refs)
        table_hbm = next(it)
        src_hbm = next(it)
        dst_hbm = next(it)
        agg_hbm = next(it)
        deg_hbm = next(it) if with_deg else None
        srcb = [next(it) for _ in range(_NIDX)]
        dstb = [next(it) for _ in range(_NIDX)]
        rows = [next(it) for _ in range(nbuf)]
        deg_v = next(it) if with_deg else None
        iota_v = next(it) if with_deg else None
        agg_sh = next(it)
        deg_sh = next(it) if with_deg else None
        sem_g = [next(it) for _ in range(nbuf)]
        sem_i = [next(it) for _ in range(_NIDX)]

        c, s, start, nch = _worker_ranges()

        def _idx_load(k, bi):
            off = (start + k) * _C
            pltpu.async_copy(src_hbm.at[pl.ds(off, _C)], srcb[bi], sem_i[bi])
            pltpu.async_copy(dst_hbm.at[pl.ds(off, _C)], dstb[bi], sem_i[bi])

        def _idx_wait(bi):
            pltpu.make_async_copy(src_hbm.at[pl.ds(0, _C)], srcb[bi],
                                  sem_i[bi]).wait()
            pltpu.make_async_copy(dst_hbm.at[pl.ds(0, _C)], dstb[bi],
                                  sem_i[bi]).wait()

        def _gather(b, bi):
            return pltpu.make_async_copy(table_hbm.at[srcb[bi]], rows[b],
                                         sem_g[b])

        _idx_load(0, 0)
        _idx_load(1, 1)
        _idx_wait(0)
        _gather(0, 0).start()

        zero16 = jnp.zeros((_L,), jnp.float32)

        zbuf = rows[nbuf - 1]

        def _zrow(i, _):
            def _zcol(j, _):
                zbuf[i, pl.ds(j * _L, _L)] = zero16
                return 0
            return lax.fori_loop(0, _D // _L, _zcol, 0)
        lax.fori_loop(0, _ZCH, _zrow, 0)

        if with_deg:
            def _zdeg(i, _):
                def _zc(j, _):
                    deg_v[i, pl.ds(j * _L, _L)] = zero16
                    return 0
                return lax.fori_loop(0, _D // _L, _zc, 0)
            lax.fori_loop(0, _DROWS, _zdeg, 0)
            for j in range(_DROWS // _L):
                iota_v[pl.ds(j * _L, _L)] = lax.iota(jnp.int32, _L) + j * _L

            @pl.when(s == 0)
            def _():
                pltpu.sync_copy(zbuf.at[pl.ds(0, _DROWS)], deg_sh)

        base_row = s * _RPT_STRIDE
        for j in range(_RPT_SPAN // _ZCH):
            pltpu.sync_copy(zbuf.at[pl.ds(0, _ZCH)],
                            agg_sh.at[pl.ds(base_row + j * _ZCH, _ZCH)])
        plsc.subcore_barrier()

        def _deg_chunk(bi):
            for j in range(_C // _L):
                d = dstb[bi][pl.ds(j * _L, _L)]
                cnt, last = plsc.scan_count(d)
                plsc.addupdate_scatter(deg_v, [d // _DCOL, lax.rem(d, _DCOL)],
                                       cnt.astype(jnp.float32), mask=last)

        def _slot(k, b, bi):
            @pl.when(k < nch)
            def _():
                @pl.when(k + 1 < nch)
                def _():
                    @pl.when(k + 2 < nch)
                    def _():
                        _idx_load(k + 2, (bi + 2) % _NIDX)
                    _idx_wait((bi + 1) % _NIDX)
                    _gather((b + 1) % nbuf, (bi + 1) % _NIDX).start()

                if with_deg:
                    _deg_chunk(bi)
                _gather(b, bi).wait()
                pltpu.sync_copy(rows[b], agg_sh.at[dstb[bi]], add=True)

        def _ring(i, _):
            for j in range(ngrp):
                k = i * ngrp + j
                _slot(k, j % nbuf, j % _NIDX)
            return 0
        lax.fori_loop(0, (_CHUNK_PER_W + 1 + ngrp - 1) // ngrp, _ring, 0)

        if with_deg:
            pltpu.sync_copy(deg_v, deg_sh.at[iota_v], add=True)
        plsc.subcore_barrier()

        pltpu.sync_copy(agg_sh.at[pl.ds(base_row, _RPT_SPAN)],
                        agg_hbm.at[c, pl.ds(base_row, _RPT_SPAN)])
        if with_deg:
            @pl.when(s == 0)
            def _():
                pltpu.sync_copy(deg_sh, deg_hbm.at[c])

    return _body


def _agg_scratch(with_deg, nbuf):
    sc = [pltpu.VMEM((_C,), jnp.int32) for _ in range(2 * _NIDX)]
    sc += [pltpu.VMEM((_C, _D), jnp.float32) for _ in range(nbuf)]
    if with_deg:
        sc += [pltpu.VMEM((_DROWS, _D), jnp.float32),
               pltpu.VMEM((_DROWS,), jnp.int32)]
    sc += [pltpu.VMEM_SHARED((_N, _D), jnp.float32)]
    if with_deg:
        sc += [pltpu.VMEM_SHARED((_DROWS, _D), jnp.float32)]
    sc += [pltpu.SemaphoreType.DMA for _ in range(nbuf + _NIDX)]
    return sc


_sc_agg_deg = pl.kernel(
    _make_agg_body(True, 2),
    out_type=[jax.ShapeDtypeStruct((_NC, _N, _D), jnp.float32),
              jax.ShapeDtypeStruct((_NC, _DROWS, _D), jnp.float32)],
    mesh=_mesh,
    scratch_types=_agg_scratch(True, 2),
    compiler_params=pltpu.CompilerParams(needs_layout_passes=False),
)

_sc_agg = pl.kernel(
    _make_agg_body(False, 3),
    out_type=jax.ShapeDtypeStruct((_NC, _N, _D), jnp.float32),
    mesh=_mesh,
    scratch_types=_agg_scratch(False, 3),
    compiler_params=pltpu.CompilerParams(needs_layout_passes=False),
)

_BN = 1000


def _normalize(agg_ref, deg_ref):
    agg = agg_ref[0] + agg_ref[1]
    deg = (deg_ref[0] + deg_ref[1])[:, :_DCOL]
    inv = 1.0 / jnp.maximum(deg, 1.0)
    inv3 = lax.broadcast_in_dim(inv, (_BN // _DCOL, _DCOL, _D), (0, 1))
    x3 = agg.reshape(_BN // _DCOL, _DCOL, _D) * inv3
    return x3.reshape(_BN, _D)


def _tc_layer1_body(agg_ref, deg_ref, w_ref, b_ref, out_ref):
    x = _normalize(agg_ref, deg_ref)
    y = jnp.dot(x, w_ref[...], preferred_element_type=jnp.float32) + b_ref[...]
    out_ref[...] = jnp.maximum(y, 0.0)


def _tc_layer2_body(agg_ref, deg_ref, w_ref, b_ref, h0_ref, out_ref):
    x = _normalize(agg_ref, deg_ref)
    y = jnp.dot(x, w_ref[...], preferred_element_type=jnp.float32) + b_ref[...]
    y = jnp.maximum(y, 0.0) + h0_ref[...]
    out_ref[...] = jnp.maximum(y, 0.0)


_agg_spec = pl.BlockSpec((_NC, _BN, _D), lambda i: (0, i, 0))
_deg_spec = pl.BlockSpec((_NC, _BN // _DCOL, _D), lambda i: (0, i, 0))
_w_spec = pl.BlockSpec((_D, _D), lambda i: (0, 0))
_b_spec = pl.BlockSpec((1, _D), lambda i: (0, 0))
_row_spec = pl.BlockSpec((_BN, _D), lambda i: (i, 0))

_tc_layer1 = pl.pallas_call(
    _tc_layer1_body,
    grid=(_N // _BN,),
    in_specs=[_agg_spec, _deg_spec, _w_spec, _b_spec],
    out_specs=_row_spec,
    out_shape=jax.ShapeDtypeStruct((_N, _D), jnp.float32),
)

_tc_layer2 = pl.pallas_call(
    _tc_layer2_body,
    grid=(_N // _BN,),
    in_specs=[_agg_spec, _deg_spec, _w_spec, _b_spec, _row_spec],
    out_specs=_row_spec,
    out_shape=jax.ShapeDtypeStruct((_N, _D), jnp.float32),
)


def kernel(h, edge_index, W1, b1, W2, b2):
    src = edge_index[0]
    dst = edge_index[1]
    agg1, deg = _sc_agg_deg(h, src, dst)
    h1 = _tc_layer1(agg1, deg, W1, b1.reshape(1, _D))
    agg2 = _sc_agg(h1, src, dst)
    return _tc_layer2(agg2, deg, W2, b2.reshape(1, _D), h)


# speedup vs baseline: 13.8266x; 1.0144x over previous
import jax
import jax.numpy as jnp
from jax import lax
from jax.experimental import pallas as pl
from jax.experimental.pallas import tpu as pltpu
from jax.experimental.pallas import tpu_sc as plsc

_N = 10000
_E = 320000
_D = 128
_L = 16
_NC = 2
_NS = 16
_NW = _NC * _NS
_C = 128
_NCHUNK = _E // _C
_CHUNK_PER_W = _NCHUNK // _NW
_CHUNK_REM = _NCHUNK - _CHUNK_PER_W * _NW
_RPT_STRIDE = 624
_RPT_SPAN = 640
_ZCH = 128
_NIDX = 4
_DROWS = 80
_DCOL = 125

_mesh = plsc.VectorSubcoreMesh(core_axis_name="c", subcore_axis_name="s")


def _worker_ranges():
    c = lax.axis_index("c")
    s = lax.axis_index("s")
    wid = c * _NS + s
    start = wid * _CHUNK_PER_W + jnp.minimum(wid, _CHUNK_REM)
    nch = jnp.where(wid < _CHUNK_REM, _CHUNK_PER_W + 1, _CHUNK_PER_W)
    return c, s, start, nch


def _make_agg_body(with_deg, nbuf):
    ngrp = 4 if nbuf == 2 else 12

    def _body(*refs):
        it = iter(refs)
        table_hbm = next(it)
        src_hbm = next(it)
        dst_hbm = next(it)
        agg_hbm = next(it)
        deg_hbm = next(it) if with_deg else None
        srcb = [next(it) for _ in range(_NIDX)]
        dstb = [next(it) for _ in range(_NIDX)]
        rows = [next(it) for _ in range(nbuf)]
        deg_v = next(it) if with_deg else None
        iota_v = next(it) if with_deg else None
        agg_sh = next(it)
        deg_sh = next(it) if with_deg else None
        sem_g = [next(it) for _ in range(nbuf)]
        sem_i = [next(it) for _ in range(_NIDX)]

        c, s, start, nch = _worker_ranges()

        def _idx_load(k, bi):
            off = (start + k) * _C
            pltpu.async_copy(src_hbm.at[pl.ds(off, _C)], srcb[bi], sem_i[bi])
            pltpu.async_copy(dst_hbm.at[pl.ds(off, _C)], dstb[bi], sem_i[bi])

        def _idx_wait(bi):
            pltpu.make_async_copy(src_hbm.at[pl.ds(0, _C)], srcb[bi],
                                  sem_i[bi]).wait()
            pltpu.make_async_copy(dst_hbm.at[pl.ds(0, _C)], dstb[bi],
                                  sem_i[bi]).wait()

        def _gather(b, bi):
            return pltpu.make_async_copy(table_hbm.at[srcb[bi]], rows[b],
                                         sem_g[b])

        _idx_load(0, 0)
        _idx_load(1, 1)
        _idx_wait(0)
        _gather(0, 0).start()

        zero16 = jnp.zeros((_L,), jnp.float32)

        zbuf = rows[nbuf - 1]

        def _zrow(i, _):
            def _zcol(j, _):
                zbuf[i, pl.ds(j * _L, _L)] = zero16
                return 0
            return lax.fori_loop(0, _D // _L, _zcol, 0)
        lax.fori_loop(0, _ZCH, _zrow, 0)

        if with_deg:
            def _zdeg(i, _):
                def _zc(j, _):
                    deg_v[i, pl.ds(j * _L, _L)] = zero16
                    return 0
                return lax.fori_loop(0, _D // _L, _zc, 0)
            lax.fori_loop(0, _DROWS, _zdeg, 0)
            for j in range(_DROWS // _L):
                iota_v[pl.ds(j * _L, _L)] = lax.iota(jnp.int32, _L) + j * _L

            @pl.when(s == 0)
            def _():
                pltpu.sync_copy(zbuf.at[pl.ds(0, _DROWS)], deg_sh)

        base_row = s * _RPT_STRIDE
        for j in range(_RPT_SPAN // _ZCH):
            pltpu.sync_copy(zbuf.at[pl.ds(0, _ZCH)],
                            agg_sh.at[pl.ds(base_row + j * _ZCH, _ZCH)])
        plsc.subcore_barrier()

        def _deg_chunk(bi):
            for j in range(_C // _L):
                d = dstb[bi][pl.ds(j * _L, _L)]
                cnt, last = plsc.scan_count(d)
                plsc.addupdate_scatter(deg_v, [d // _DCOL, lax.rem(d, _DCOL)],
                                       cnt.astype(jnp.float32), mask=last)

        def _slot(k, b, bi):
            @pl.when(k < nch)
            def _():
                @pl.when(k + 1 < nch)
                def _():
                    @pl.when(k + 2 < nch)
                    def _():
                        _idx_load(k + 2, (bi + 2) % _NIDX)
                    _idx_wait((bi + 1) % _NIDX)
                    _gather((b + 1) % nbuf, (bi + 1) % _NIDX).start()

                if with_deg:
                    _deg_chunk(bi)
                _gather(b, bi).wait()
                pltpu.sync_copy(rows[b], agg_sh.at[dstb[bi]], add=True)

        def _ring(i, _):
            for j in range(ngrp):
                k = i * ngrp + j
                _slot(k, j % nbuf, j % _NIDX)
            return 0
        lax.fori_loop(0, (_CHUNK_PER_W + 1 + ngrp - 1) // ngrp, _ring, 0)

        if with_deg:
            pltpu.sync_copy(deg_v, deg_sh.at[iota_v], add=True)
        plsc.subcore_barrier()

        pltpu.sync_copy(agg_sh.at[pl.ds(base_row, _RPT_SPAN)],
                        agg_hbm.at[c, pl.ds(base_row, _RPT_SPAN)])
        if with_deg:
            @pl.when(s == 0)
            def _():
                pltpu.sync_copy(deg_sh, deg_hbm.at[c])

    return _body


def _agg_scratch(with_deg, nbuf):
    sc = [pltpu.VMEM((_C,), jnp.int32) for _ in range(2 * _NIDX)]
    sc += [pltpu.VMEM((_C, _D), jnp.float32) for _ in range(nbuf)]
    if with_deg:
        sc += [pltpu.VMEM((_DROWS, _D), jnp.float32),
               pltpu.VMEM((_DROWS,), jnp.int32)]
    sc += [pltpu.VMEM_SHARED((_N, _D), jnp.float32)]
    if with_deg:
        sc += [pltpu.VMEM_SHARED((_DROWS, _D), jnp.float32)]
    sc += [pltpu.SemaphoreType.DMA for _ in range(nbuf + _NIDX)]
    return sc


_sc_agg_deg = pl.kernel(
    _make_agg_body(True, 2),
    out_type=[jax.ShapeDtypeStruct((_NC, _N, _D), jnp.float32),
              jax.ShapeDtypeStruct((_NC, _DROWS, _D), jnp.float32)],
    mesh=_mesh,
    scratch_types=_agg_scratch(True, 2),
    compiler_params=pltpu.CompilerParams(needs_layout_passes=False),
)

_sc_agg = pl.kernel(
    _make_agg_body(False, 3),
    out_type=jax.ShapeDtypeStruct((_NC, _N, _D), jnp.float32),
    mesh=_mesh,
    scratch_types=_agg_scratch(False, 3),
    compiler_params=pltpu.CompilerParams(needs_layout_passes=False),
)

_BN = 2000


def _normalize(agg_ref, deg_ref):
    agg = agg_ref[0] + agg_ref[1]
    deg = (deg_ref[0] + deg_ref[1])[:, :_DCOL]
    inv = 1.0 / jnp.maximum(deg, 1.0)
    inv3 = lax.broadcast_in_dim(inv, (_BN // _DCOL, _DCOL, _D), (0, 1))
    x3 = agg.reshape(_BN // _DCOL, _DCOL, _D) * inv3
    return x3.reshape(_BN, _D)


def _tc_layer1_body(agg_ref, deg_ref, w_ref, b_ref, out_ref):
    x = _normalize(agg_ref, deg_ref)
    y = jnp.dot(x, w_ref[...], preferred_element_type=jnp.float32) + b_ref[...]
    out_ref[...] = jnp.maximum(y, 0.0)


def _tc_layer2_body(agg_ref, deg_ref, w_ref, b_ref, h0_ref, out_ref):
    x = _normalize(agg_ref, deg_ref)
    y = jnp.dot(x, w_ref[...], preferred_element_type=jnp.float32) + b_ref[...]
    y = jnp.maximum(y, 0.0) + h0_ref[...]
    out_ref[...] = jnp.maximum(y, 0.0)


_agg_spec = pl.BlockSpec((_NC, _BN, _D), lambda i: (0, i, 0))
_deg_spec = pl.BlockSpec((_NC, _BN // _DCOL, _D), lambda i: (0, i, 0))
_w_spec = pl.BlockSpec((_D, _D), lambda i: (0, 0))
_b_spec = pl.BlockSpec((1, _D), lambda i: (0, 0))
_row_spec = pl.BlockSpec((_BN, _D), lambda i: (i, 0))

_tc_layer1 = pl.pallas_call(
    _tc_layer1_body,
    grid=(_N // _BN,),
    in_specs=[_agg_spec, _deg_spec, _w_spec, _b_spec],
    out_specs=_row_spec,
    out_shape=jax.ShapeDtypeStruct((_N, _D), jnp.float32),
)

_tc_layer2 = pl.pallas_call(
    _tc_layer2_body,
    grid=(_N // _BN,),
    in_specs=[_agg_spec, _deg_spec, _w_spec, _b_spec, _row_spec],
    out_specs=_row_spec,
    out_shape=jax.ShapeDtypeStruct((_N, _D), jnp.float32),
)


def kernel(h, edge_index, W1, b1, W2, b2):
    src = edge_index[0]
    dst = edge_index[1]
    agg1, deg = _sc_agg_deg(h, src, dst)
    h1 = _tc_layer1(agg1, deg, W1, b1.reshape(1, _D))
    agg2 = _sc_agg(h1, src, dst)
    return _tc_layer2(agg2, deg, W2, b2.reshape(1, _D), h)
